# Initial kernel scaffold; baseline (speedup 1.0000x reference)
import jax
import jax.numpy as jnp
from jax.experimental import pallas as pl


def kernel(x, token_table, pos_table):
    raise NotImplementedError("write your pallas kernel here")


# speedup vs baseline: 2.2641x; 2.2641x over previous
import functools

import jax
import jax.numpy as jnp
from jax import lax
from jax.experimental import pallas as pl
from jax.experimental.pallas import tpu as pltpu
from jax.experimental.pallas import tpu_sc as plsc

_B, _L, _D = 1024, 200, 128
_H = _L // 2
_NC, _NS = 2, 16
_NW = _NC * _NS
_SPW = _B // _NW
_LANES = 16

_mesh = plsc.VectorSubcoreMesh(
    core_axis_name="c", subcore_axis_name="s", num_cores=_NC, num_subcores=_NS
)


@functools.partial(
    pl.kernel,
    out_type=jax.ShapeDtypeStruct((_B, 2, _H, _D), jnp.float32),
    mesh=_mesh,
    scratch_types=[
        pltpu.VMEM((2, _H), jnp.int32),
        pltpu.VMEM((2, _H, _D), jnp.float32),
        pltpu.VMEM((2, _H, _D), jnp.float32),
        pltpu.SemaphoreType.DMA,
    ],
)
def _emb(x_hbm, tab_hbm, pos_hbm, out_hbm, idx_v, rows_v, pos_v, sem):
    wid = lax.axis_index("s") * _NC + lax.axis_index("c")
    pltpu.sync_copy(pos_hbm, pos_v)

    def seq_body(s, carry):
        b = wid * _SPW + s
        pltpu.sync_copy(x_hbm.at[b], idx_v)
        g0 = pltpu.async_copy(tab_hbm.at[idx_v.at[0]], rows_v.at[0], sem)
        g1 = pltpu.async_copy(tab_hbm.at[idx_v.at[1]], rows_v.at[1], sem)
        g0.wait()
        g1.wait()

        def add_body(r, cc):
            for j in range(2):
                for k in range(_D // _LANES):
                    sl = pl.ds(k * _LANES, _LANES)
                    plsc.addupdate(rows_v.at[j, r, sl], pos_v[j, r, sl])
            return cc

        lax.fori_loop(0, _H, add_body, 0, unroll=2)
        pltpu.sync_copy(rows_v, out_hbm.at[b])
        return carry

    lax.fori_loop(0, _SPW, seq_body, 0)


def kernel(x, token_table, pos_table):
    x3 = x.reshape(_B, 2, _H)
    pos3 = pos_table.reshape(2, _H, _D)
    out = _emb(x3, token_table, pos3)
    return out.reshape(_B, _L, _D)


# speedup vs baseline: 3.4889x; 1.5410x over previous
import functools

import jax
import jax.numpy as jnp
from jax import lax
from jax.experimental import pallas as pl
from jax.experimental.pallas import tpu as pltpu
from jax.experimental.pallas import tpu_sc as plsc

_B, _L, _D = 1024, 200, 128
_CH = 100
_NCHUNK = (_B * _L) // _CH
_NC, _NS = 2, 16
_NW = _NC * _NS
_SEC = _NCHUNK // _NW
_NBUF = 4
_LANES = 16

_mesh = plsc.VectorSubcoreMesh(
    core_axis_name="c", subcore_axis_name="s", num_cores=_NC, num_subcores=_NS
)


@functools.partial(
    pl.kernel,
    out_type=jax.ShapeDtypeStruct((_NCHUNK, _CH, _D), jnp.float32),
    mesh=_mesh,
    scratch_types=[
        pltpu.VMEM((_SEC, _CH), jnp.int32),
        pltpu.VMEM((_L, _D), jnp.float32),
        [pltpu.VMEM((_CH, _D), jnp.float32) for _ in range(_NBUF)],
        [pltpu.SemaphoreType.DMA for _ in range(_NBUF)],
        [pltpu.SemaphoreType.DMA for _ in range(_NBUF)],
    ],
)
def _emb(x_hbm, tab_hbm, pos_hbm, out_hbm, idx_v, pos_v, bufs, gsem, wsem):
    wid = lax.axis_index("s") * _NC + lax.axis_index("c")
    base = wid * _SEC
    pltpu.sync_copy(x_hbm.at[pl.ds(base, _SEC)], idx_v)
    pltpu.sync_copy(pos_hbm, pos_v)

    def issue_gather(s):
        b = s % _NBUF
        return pltpu.async_copy(tab_hbm.at[idx_v.at[s]], bufs[b], gsem[b])

    def issue_write(s):
        b = s % _NBUF
        return pltpu.async_copy(bufs[b], out_hbm.at[base + s], wsem[b])

    def add_pos(s):
        b = s % _NBUF
        off = (s % 2) * _CH

        def body(r, carry):
            for k in range(_D // _LANES):
                sl = pl.ds(k * _LANES, _LANES)
                plsc.addupdate(bufs[b].at[r, sl], pos_v[off + r, sl])
            return carry

        lax.fori_loop(0, _CH, body, 0, unroll=2)

    gathers = [None] * _SEC
    writes = [None] * _SEC
    gathers[0] = issue_gather(0)
    gathers[1] = issue_gather(1)
    for s in range(_SEC):
        gathers[s].wait()
        add_pos(s)
        writes[s] = issue_write(s)
        if s + 2 < _SEC:
            if s - 2 >= 0:
                writes[s - 2].wait()
            gathers[s + 2] = issue_gather(s + 2)
    writes[_SEC - 2].wait()
    writes[_SEC - 1].wait()


def kernel(x, token_table, pos_table):
    x2 = x.reshape(_NCHUNK, _CH)
    out = _emb(x2, token_table, pos_table)
    return out.reshape(_B, _L, _D)
